# Initial kernel scaffold; baseline (speedup 1.0000x reference)
import jax
import jax.numpy as jnp
from jax.experimental import pallas as pl


def kernel(ticker_indices, table):
    raise NotImplementedError("write your pallas kernel here")


# speedup vs baseline: 1.0932x; 1.0932x over previous
import functools

import jax
import jax.numpy as jnp
from jax import lax
from jax.experimental import pallas as pl
from jax.experimental.pallas import tpu as pltpu
from jax.experimental.pallas import tpu_sc as plsc


def _make_gather(B: int, D: int, num_workers: int, C: int):
    b_per_w = B // num_workers
    nchunks = b_per_w // C
    mesh = plsc.VectorSubcoreMesh(core_axis_name="c", subcore_axis_name="s")
    NC = 2

    @functools.partial(
        pl.kernel,
        mesh=mesh,
        compiler_params=pltpu.CompilerParams(use_tc_tiling_on_sc=False),
        out_type=jax.ShapeDtypeStruct((B, D), jnp.float32),
        scratch_types=[
            pltpu.VMEM((C,), jnp.int32),
            pltpu.VMEM((C, D), jnp.float32),
            pltpu.SemaphoreType.DMA,
        ],
    )
    def gather_kernel(idx_hbm, table_hbm, out_hbm, idx_v, rows_v, sem):
        wid = lax.axis_index("s") * NC + lax.axis_index("c")
        base = wid * b_per_w

        def body(c, carry):
            off = pl.multiple_of(base + c * C, C)
            pltpu.sync_copy(idx_hbm.at[pl.ds(off, C)], idx_v)
            pltpu.async_copy(table_hbm.at[idx_v], rows_v, sem).wait()
            pltpu.sync_copy(rows_v, out_hbm.at[pl.ds(off, C)])
            return carry

        lax.fori_loop(0, nchunks, body, 0)

    return gather_kernel


def kernel(ticker_indices, table):
    batch, hist = ticker_indices.shape
    _, d = table.shape
    B = batch * hist
    flat_idx = ticker_indices.reshape(B).astype(jnp.int32)
    out = _make_gather(B, d, num_workers=32, C=1024)(flat_idx, table)
    return out.reshape(batch, hist, d)


# speedup vs baseline: 1.7790x; 1.6272x over previous
import functools

import jax
import jax.numpy as jnp
from jax import lax
from jax.experimental import pallas as pl
from jax.experimental.pallas import tpu as pltpu
from jax.experimental.pallas import tpu_sc as plsc


def _make_gather(BATCH: int, HIST: int, D: int, num_workers: int, RB: int):
    b_per_w = BATCH // num_workers
    nchunks = b_per_w // RB
    C = RB * HIST
    mesh = plsc.VectorSubcoreMesh(core_axis_name="c", subcore_axis_name="s")
    NC = 2

    @functools.partial(
        pl.kernel,
        mesh=mesh,
        compiler_params=pltpu.CompilerParams(use_tc_tiling_on_sc=False),
        out_type=jax.ShapeDtypeStruct((BATCH, HIST, D), jnp.float32),
        scratch_types=[
            pltpu.VMEM((RB, HIST), jnp.int32),
            pltpu.VMEM((RB, HIST, D), jnp.float32),
            pltpu.SemaphoreType.DMA,
            pltpu.SemaphoreType.DMA,
        ],
    )
    def gather_kernel(idx_hbm, table_hbm, out_hbm, idx_v, rows_v, gsem, ssem):
        wid = lax.axis_index("s") * NC + lax.axis_index("c")
        batch_base = wid * b_per_w

        def body(c, carry):
            b0 = pl.multiple_of(batch_base + c * RB, RB)
            pltpu.sync_copy(idx_hbm.at[pl.ds(b0, RB)], idx_v)
            for j in range(RB):
                pltpu.async_copy(table_hbm.at[idx_v.at[j]], rows_v.at[j], gsem)
            for j in range(RB):
                pltpu.make_async_copy(
                    table_hbm.at[idx_v.at[j]], rows_v.at[j], gsem
                ).wait()
            pltpu.async_copy(rows_v, out_hbm.at[pl.ds(b0, RB)], ssem).wait()
            return carry

        lax.fori_loop(0, nchunks, body, 0)

    return gather_kernel


def kernel(ticker_indices, table):
    batch, hist = ticker_indices.shape
    _, d = table.shape
    out = _make_gather(batch, hist, d, num_workers=32, RB=32)(
        ticker_indices.astype(jnp.int32), table
    )
    return out


# speedup vs baseline: 1.7956x; 1.0093x over previous
import functools

import jax
import jax.numpy as jnp
from jax import lax
from jax.experimental import pallas as pl
from jax.experimental.pallas import tpu as pltpu
from jax.experimental.pallas import tpu_sc as plsc


def _make_gather(BATCH: int, HIST: int, D: int, num_workers: int, RB: int):
    b_per_w = BATCH // num_workers
    nchunks = b_per_w // RB
    assert nchunks % 2 == 0
    mesh = plsc.VectorSubcoreMesh(core_axis_name="c", subcore_axis_name="s")
    NC = 2

    @functools.partial(
        pl.kernel,
        mesh=mesh,
        compiler_params=pltpu.CompilerParams(use_tc_tiling_on_sc=False),
        out_type=jax.ShapeDtypeStruct((BATCH, HIST, D), jnp.float32),
        scratch_types=[
            pltpu.VMEM((b_per_w, HIST), jnp.int32),
            pltpu.VMEM((RB, HIST, D), jnp.float32),
            pltpu.VMEM((RB, HIST, D), jnp.float32),
            pltpu.SemaphoreType.DMA,
            pltpu.SemaphoreType.DMA,
            pltpu.SemaphoreType.DMA,
        ],
    )
    def gather_kernel(
        idx_hbm, table_hbm, out_hbm, idx_v, rows0, rows1, gsem, ssem0, ssem1
    ):
        wid = lax.axis_index("s") * NC + lax.axis_index("c")
        batch_base = wid * b_per_w
        pltpu.sync_copy(idx_hbm.at[pl.ds(batch_base, b_per_w)], idx_v)

        bufs = (rows0, rows1)
        ssems = (ssem0, ssem1)

        def chunk(c, rows_v, ssem):
            @pl.when(c >= 2)
            def _():
                pltpu.make_async_copy(
                    rows_v,
                    out_hbm.at[pl.ds(batch_base + (c - 2) * RB, RB)],
                    ssem,
                ).wait()

            for j in range(RB):
                pltpu.async_copy(
                    table_hbm.at[idx_v.at[c * RB + j]], rows_v.at[j], gsem
                )
            for j in range(RB):
                pltpu.make_async_copy(
                    table_hbm.at[idx_v.at[c * RB + j]], rows_v.at[j], gsem
                ).wait()
            pltpu.async_copy(
                rows_v, out_hbm.at[pl.ds(batch_base + c * RB, RB)], ssem
            )

        def body(i, carry):
            chunk(i * 2, rows0, ssem0)
            chunk(i * 2 + 1, rows1, ssem1)
            return carry

        lax.fori_loop(0, nchunks // 2, body, 0)
        for p in range(2):
            c = nchunks - 2 + p
            pltpu.make_async_copy(
                bufs[p], out_hbm.at[pl.ds(batch_base + c * RB, RB)], ssems[p]
            ).wait()

    return gather_kernel


def kernel(ticker_indices, table):
    batch, hist = ticker_indices.shape
    _, d = table.shape
    out = _make_gather(batch, hist, d, num_workers=32, RB=32)(
        ticker_indices.astype(jnp.int32), table
    )
    return out
